# baseline (speedup 1.0000x reference)
import jax
import jax.numpy as jnp
from jax.experimental import pallas as pl

_N = 256
_B = 32
_P2 = _B // 2
_SEQ = 12
_HOR = 12
_U = 64
_M = 3
_F32 = jnp.float32


def _to_wide(h):
    return jnp.concatenate(
        [h[p * _N:(p + 1) * _N, :] for p in range(_P2)], axis=1)


def _to_pm(w, piece=2 * _U):
    return jnp.concatenate(
        [w[:, p * piece:(p + 1) * piece] for p in range(_P2)], axis=0)


def _cheb_wide(scat, xw):
    y = jnp.dot(scat, xw, preferred_element_type=_F32)
    return y[:_N], y[_N:]


def _parts_cat(p0, w1, w2):
    return jnp.concatenate([p0, _to_pm(w1), _to_pm(w2)], axis=1)


def _cell0(scat, xk, hcat, hpm, wgh, bg, hgk, wch, bc, hck):
    cg = jnp.dot(xk, hgk, preferred_element_type=_F32)
    val = (jnp.dot(hcat, wgh, preferred_element_type=_F32)
           + _to_pm(cg, 4 * _U) + bg)
    val = jax.nn.sigmoid(val)
    r = val[:, :2 * _U]
    u = val[:, 2 * _U:]
    rh = r * hpm
    z1, z2 = _cheb_wide(scat, _to_wide(rh))
    rcat = _parts_cat(rh, z1, z2)
    cc = jnp.dot(xk, hck, preferred_element_type=_F32)
    c = jnp.tanh(jnp.dot(rcat, wch, preferred_element_type=_F32)
                 + _to_pm(cc) + bc)
    hn = u * hpm + (1.0 - u) * c
    return hn, _to_wide(hn)


def _cell1(scat, xcat, hcat, hpm, wg, bg, wc, bc):
    val = jnp.dot(jnp.concatenate([xcat, hcat], axis=1), wg,
                  preferred_element_type=_F32) + bg
    val = jax.nn.sigmoid(val)
    r = val[:, :2 * _U]
    u = val[:, 2 * _U:]
    rh = r * hpm
    z1, z2 = _cheb_wide(scat, _to_wide(rh))
    rcat = _parts_cat(rh, z1, z2)
    c = jnp.tanh(jnp.dot(jnp.concatenate([xcat, rcat], axis=1), wc,
                         preferred_element_type=_F32) + bc)
    hn = u * hpm + (1.0 - u) * c
    return hn, _to_wide(hn)


def _cheb_dual(scat, h0w, h1w):
    y = jnp.dot(scat, jnp.concatenate([h0w, h1w], axis=1),
                preferred_element_type=_F32)
    w = _B * _U
    return (y[:_N, :w], y[_N:, :w]), (y[:_N, w:], y[_N:, w:])


def _xk(scat, xc):
    y = jnp.dot(scat, xc, preferred_element_type=_F32)
    return jnp.concatenate([xc, y[:_N], y[_N:]], axis=1)


def _dcrnn_body(scat_ref, x_ref, pwc_ref,
                e0wgh, e0bg, e0hgk, e0wch, e0bc, e0hck,
                e1wg, e1bg, e1wc, e1bc,
                d0wgh, d0bg, d0hgk, d0wch, d0bc, d0hck,
                d1wg, d1bg, d1wc, d1bc,
                pb_ref, out_ref):
    scat = scat_ref[...]
    pwc = pwc_ref[...]
    e0 = (e0wgh[...], e0bg[...], e0hgk[...], e0wch[...], e0bc[...], e0hck[...])
    e1 = (e1wg[...], e1bg[...], e1wc[...], e1bc[...])
    d0 = (d0wgh[...], d0bg[...], d0hgk[...], d0wch[...], d0bc[...], d0hck[...])
    d1 = (d1wg[...], d1bg[...], d1wc[...], d1bc[...])
    pb = pb_ref[...]

    def enc_step(t, carry):
        h0p, h0w, h1p, h1w = carry
        xc = x_ref[pl.ds(t * _N, _N), :]
        xk = _xk(scat, xc)
        (a1, a2), (b1, b2) = _cheb_dual(scat, h0w, h1w)
        h1cat = _parts_cat(h1p, b1, b2)
        h0p, h0w = _cell0(scat, xk, _parts_cat(h0p, a1, a2), h0p, *e0)
        y1, y2 = _cheb_wide(scat, h0w)
        xcat1 = _parts_cat(h0p, y1, y2)
        h1p, h1w = _cell1(scat, xcat1, h1cat, h1p, *e1)
        return h0p, h0w, h1p, h1w

    hp0 = jnp.zeros((_P2 * _N, 2 * _U), _F32)
    hw0 = jnp.zeros((_N, _B * _U), _F32)
    h0p, h0w, h1p, h1w = jax.lax.fori_loop(
        0, _SEQ, enc_step, (hp0, hw0, hp0, hw0))

    def dec_step(t, carry):
        h0p, h0w, h1p, h1w, oc = carry
        xk = _xk(scat, oc)
        (a1, a2), (b1, b2) = _cheb_dual(scat, h0w, h1w)
        h1cat = _parts_cat(h1p, b1, b2)
        h0p, h0w = _cell0(scat, xk, _parts_cat(h0p, a1, a2), h0p, *d0)
        y1, y2 = _cheb_wide(scat, h0w)
        xcat1 = _parts_cat(h0p, y1, y2)
        h1p, h1w = _cell1(scat, xcat1, h1cat, h1p, *d1)
        oc = jnp.dot(h1w, pwc, preferred_element_type=_F32) + pb
        out_ref[pl.ds(t * _N, _N), :] = oc
        return h0p, h0w, h1p, h1w, oc

    oc0 = jnp.zeros((_N, _B), _F32)
    jax.lax.fori_loop(0, _HOR, dec_step, (h0p, h0w, h1p, h1w, oc0))


def _blkdiag2(w):
    z = jnp.zeros_like(w)
    return jnp.concatenate([jnp.concatenate([w, z], axis=1),
                            jnp.concatenate([z, w], axis=1)], axis=0)


def _gate_major(w):
    perm = []
    for gate in range(2):
        for lsb in range(2):
            base = lsb * 2 * _U + gate * _U
            perm.extend(range(base, base + _U))
    return w[:, jnp.array(perm)]


def _prep0(p):
    wg = p["Wg"].reshape(1 + _U, _M, 2 * _U)
    wc = p["Wc"].reshape(1 + _U, _M, _U)
    whg = jnp.transpose(wg[1:], (1, 0, 2))
    whc = jnp.transpose(wc[1:], (1, 0, 2))
    wgh_cat = _gate_major(
        jnp.concatenate([_blkdiag2(whg[m]) for m in range(_M)], axis=0))
    wch_cat = jnp.concatenate([_blkdiag2(whc[m]) for m in range(_M)], axis=0)
    eye = jnp.eye(_B, dtype=_F32)
    hgk = jnp.concatenate(
        [(eye[:, :, None] * wg[0, m][None, None, :]).reshape(_B, _B * 2 * _U)
         for m in range(_M)], axis=0)
    hgk = hgk.reshape(_M * _B, _P2, 2, 2, _U)
    hgk = jnp.transpose(hgk, (0, 1, 3, 2, 4)).reshape(_M * _B, _B * 2 * _U)
    hck = jnp.concatenate(
        [(eye[:, :, None] * wc[0, m][None, None, :]).reshape(_B, _B * _U)
         for m in range(_M)], axis=0)
    bg2 = _gate_major(jnp.tile(p["bg"], 2).reshape(1, 4 * _U))
    bc2 = jnp.tile(p["bc"], 2).reshape(1, 2 * _U)
    return (wgh_cat, bg2, hgk, wch_cat, bc2, hck)


def _prep1(p):
    wg = p["Wg"].reshape(2 * _U, _M, 2 * _U)
    wc = p["Wc"].reshape(2 * _U, _M, _U)
    wxg = jnp.transpose(wg[:_U], (1, 0, 2))
    wxc = jnp.transpose(wc[:_U], (1, 0, 2))
    whg = jnp.transpose(wg[_U:], (1, 0, 2))
    whc = jnp.transpose(wc[_U:], (1, 0, 2))
    wg_cat = _gate_major(jnp.concatenate(
        [_blkdiag2(wxg[m]) for m in range(_M)]
        + [_blkdiag2(whg[m]) for m in range(_M)], axis=0))
    wc_cat = jnp.concatenate(
        [_blkdiag2(wxc[m]) for m in range(_M)]
        + [_blkdiag2(whc[m]) for m in range(_M)], axis=0)
    bg2 = _gate_major(jnp.tile(p["bg"], 2).reshape(1, 4 * _U))
    bc2 = jnp.tile(p["bc"], 2).reshape(1, 2 * _U)
    return (wg_cat, bg2, wc_cat, bc2)


def kernel(inputs, params, adj):
    x_c = jnp.transpose(inputs, (0, 2, 1)).reshape(_SEQ * _N, _B)
    dsum = jnp.sum(adj, axis=1)
    dis = jnp.where(dsum > 0, 1.0 / jnp.sqrt(dsum), 0.0)
    s1 = -(dis[:, None] * adj * dis[None, :])
    s2 = 2.0 * (s1 @ s1) - jnp.eye(_N, dtype=_F32)
    scat = jnp.concatenate([s1, s2], axis=0)
    pwc = (jnp.eye(_B, dtype=_F32)[:, None, :]
           * params["proj"]["W"][None, :, 0:1]).reshape(_B * _U, _B)
    args = [scat, x_c, pwc]
    args.extend(_prep0(params["enc"][0]))
    args.extend(_prep1(params["enc"][1]))
    args.extend(_prep0(params["dec"][0]))
    args.extend(_prep1(params["dec"][1]))
    args.append(params["proj"]["b"].reshape(1, 1))
    out = pl.pallas_call(
        _dcrnn_body,
        out_shape=jax.ShapeDtypeStruct((_HOR * _N, _B), _F32),
    )(*args)
    out = out.reshape(_HOR, _N, _B)
    return jnp.transpose(out, (0, 2, 1))
